# baseline (speedup 1.0000x reference)
import functools

import jax
import jax.numpy as jnp
from jax import lax
from jax.experimental import pallas as pl
from jax.experimental.pallas import tpu as pltpu
from jax.experimental.pallas import tpu_sc as plsc

_PRE_T = 0.05
_NMS_T = 0.6
_K = 1000
_KP = 1024
_RP = 104
_OUT = 100

_N = 20000
_NT = 20480
_TILES = 16
_E = _NT // _TILES
_CH = _E // 16
_RUN = _E + 16


def _sc_select_body(scores_hbm, table_hbm, out_hbm,
                    sc_v, key_v, hist_v, allh_v, cnt_v, gt_v, eq_v,
                    sel_v, selg_v, rows_v,
                    sh_hist, sh_cnt, sh_gt, sh_eq, sh_sel, sem):
    sid = lax.axis_index("s")
    base = sid * _E
    lanes = lax.iota(jnp.int32, 16)
    ones_i = jnp.ones((16,), jnp.int32)

    pltpu.sync_copy(scores_hbm.at[pl.ds(base, _E)], sc_v)

    def keys_body(c, _):
        s = sc_v[pl.ds(c * 16, 16)]
        st = jnp.where(s > _PRE_T, s, -1.0)
        k = plsc.bitcast(st, jnp.uint32) ^ jnp.uint32(0x80000000)
        key_v[pl.ds(c * 16, 16)] = k
        return 0

    lax.fori_loop(0, _CH, keys_body, 0)

    rem = jnp.int32(_K)
    prefix = jnp.uint32(0)
    for shift in (24, 16, 8, 0):
        def clr(i, _):
            hist_v[pl.ds(i * 16, 16)] = jnp.zeros((16,), jnp.int32)
            return 0

        lax.fori_loop(0, 16, clr, 0)

        def hb(c, _, shift=shift, prefix=prefix):
            k = key_v[pl.ds(c * 16, 16)]
            digit = ((k >> shift) & jnp.uint32(0xFF)).astype(jnp.int32)
            if shift == 24:
                plsc.addupdate_scatter(hist_v, [digit], ones_i)
            else:
                pm = (k >> (shift + 8)) == (prefix >> (shift + 8))
                plsc.addupdate_scatter(hist_v, [digit], ones_i, mask=pm)
            return 0

        lax.fori_loop(0, _CH, hb, 0)

        pltpu.sync_copy(hist_v, sh_hist.at[sid])
        plsc.subcore_barrier()
        pltpu.sync_copy(sh_hist, allh_v)

        def mg(c, _):
            acc = jnp.zeros((16,), jnp.int32)
            for t in range(_TILES):
                acc = acc + allh_v[t, pl.ds(c * 16, 16)]
            hist_v[pl.ds(c * 16, 16)] = acc
            return 0

        lax.fori_loop(0, 16, mg, 0)
        plsc.subcore_barrier()

        def scan(i, st, rem=rem):
            found, dstar, above = st
            cc = 15 - i
            chunk = hist_v[pl.ds(cc * 16, 16)]
            rev = lax.rev(chunk, (0,))
            cum = jnp.cumsum(rev) + above
            hitv = cum >= rem
            nhit = jnp.sum(hitv.astype(jnp.int32))
            f0 = jnp.min(jnp.where(hitv, lanes, 16))
            excl = cum - rev
            excl_at = jnp.sum(jnp.where(lanes == f0, excl, 0))
            d_new = cc * 16 + 15 - f0
            take = (nhit > 0) & (found == 0)
            found2 = jnp.where(take, 1, found)
            dstar2 = jnp.where(take, d_new, dstar)
            above2 = jnp.where(
                take, excl_at,
                jnp.where(found == 0, above + jnp.sum(chunk), above))
            return (found2, dstar2, above2)

        _, dstar, cnt_gt = lax.fori_loop(
            0, 16, scan, (jnp.int32(0), jnp.int32(0), jnp.int32(0)))
        rem = rem - cnt_gt
        prefix = prefix | (dstar.astype(jnp.uint32) << shift)

    vkey = prefix

    def cp(c, st):
        pg, pe = st
        kk = key_v[pl.ds(c * 16, 16)]
        gidx = base + c * 16 + lanes
        m_gt = kk > vkey
        m_eq = kk == vkey
        plsc.store_compressed(gt_v.at[pl.ds(pg, 16)], gidx, mask=m_gt)
        plsc.store_compressed(eq_v.at[pl.ds(pe, 16)], gidx, mask=m_eq)
        return (pg + jnp.sum(m_gt.astype(jnp.int32)),
                pe + jnp.sum(m_eq.astype(jnp.int32)))

    n_gt, n_eq = lax.fori_loop(0, _CH, cp, (jnp.int32(0), jnp.int32(0)))
    cnt_v[...] = jnp.where(lanes == 0, n_gt, jnp.where(lanes == 1, n_eq, 0))
    pltpu.sync_copy(cnt_v, sh_cnt.at[sid])
    pltpu.sync_copy(gt_v, sh_gt.at[sid])
    pltpu.sync_copy(eq_v, sh_eq.at[sid])
    plsc.subcore_barrier()

    @pl.when(sid == 0)
    def _():
        def gt_t(t, pos):
            pltpu.sync_copy(sh_cnt.at[t], cnt_v)
            n = cnt_v[pl.ds(0, 16)][0]
            pltpu.sync_copy(sh_gt.at[t], gt_v)

            def inner(c, _):
                sel_v[pl.ds(pos + c * 16, 16)] = gt_v[pl.ds(c * 16, 16)]
                return 0

            lax.fori_loop(0, (n + 15) // 16, inner, 0)
            return pos + n

        total_gt = lax.fori_loop(0, _TILES, gt_t, jnp.int32(0))

        def eq_t(t, st):
            pos, need = st
            pltpu.sync_copy(sh_cnt.at[t], cnt_v)
            ne = cnt_v[pl.ds(0, 16)][1]
            take = jnp.minimum(ne, need)
            pltpu.sync_copy(sh_eq.at[t], eq_v)

            def inner(c, _):
                sel_v[pl.ds(pos + c * 16, 16)] = eq_v[pl.ds(c * 16, 16)]
                return 0

            lax.fori_loop(0, (take + 15) // 16, inner, 0)
            return (pos + take, need - take)

        lax.fori_loop(0, _TILES, eq_t, (total_gt, rem))
        sel_v[pl.ds(_K, 16)] = _N + lanes
        sel_v[pl.ds(_K + 8, 16)] = _N + 8 + lanes
        pltpu.sync_copy(sel_v.at[pl.ds(0, _KP)], sh_sel)

    plsc.subcore_barrier()

    pltpu.sync_copy(sh_sel.at[pl.ds(sid * 64, 64)], selg_v)
    pltpu.async_copy(table_hbm.at[selg_v], rows_v, sem).wait()
    pltpu.sync_copy(rows_v, out_hbm.at[pl.ds(sid * 64, 64)])


_sc_select = functools.partial(
    pl.kernel,
    mesh=plsc.VectorSubcoreMesh(
        core_axis_name="c", subcore_axis_name="s", num_cores=1),
    compiler_params=pltpu.CompilerParams(
        needs_layout_passes=False, use_tc_tiling_on_sc=False),
    out_type=jax.ShapeDtypeStruct((_KP, 16), jnp.float32),
    scratch_types=[
        pltpu.VMEM((_E,), jnp.float32),
        pltpu.VMEM((_E,), jnp.uint32),
        pltpu.VMEM((256,), jnp.int32),
        pltpu.VMEM((_TILES, 256), jnp.int32),
        pltpu.VMEM((16,), jnp.int32),
        pltpu.VMEM((_RUN,), jnp.int32),
        pltpu.VMEM((_RUN,), jnp.int32),
        pltpu.VMEM((_KP + 16,), jnp.int32),
        pltpu.VMEM((64,), jnp.int32),
        pltpu.VMEM((64, 16), jnp.float32),
        pltpu.VMEM_SHARED((_TILES, 256), jnp.int32),
        pltpu.VMEM_SHARED((_TILES, 16), jnp.int32),
        pltpu.VMEM_SHARED((_TILES, _RUN), jnp.int32),
        pltpu.VMEM_SHARED((_TILES, _RUN), jnp.int32),
        pltpu.VMEM_SHARED((_KP,), jnp.int32),
        pltpu.SemaphoreType.DMA,
    ],
)(_sc_select_body)


def _nms_body(pr_ref, pc_ref, bx_ref, out_ref):
    m = jnp.max(bx_ref[...]) + 1.0
    x1r = pr_ref[0:1, :]
    y1r = pr_ref[1:2, :]
    x2r = pr_ref[2:3, :]
    y2r = pr_ref[3:4, :]
    rsr = pr_ref[4:5, :]
    lr = pr_ref[5:6, :]
    ixr = pr_ref[6:7, :]
    x1c = pc_ref[:, 0:1]
    y1c = pc_ref[:, 1:2]
    x2c = pc_ref[:, 2:3]
    y2c = pc_ref[:, 3:4]
    rsc = pc_ref[:, 4:5]
    lc = pc_ref[:, 5:6]
    ixc = pc_ref[:, 6:7]

    sr = jnp.where(rsr > _PRE_T, rsr, -1.0)
    scl = jnp.where(rsc > _PRE_T, rsc, -1.0)

    offr = lr * m
    offc = lc * m
    ox1r = x1r + offr
    oy1r = y1r + offr
    ox2r = x2r + offr
    oy2r = y2r + offr
    ox1c = x1c + offc
    oy1c = y1c + offc
    ox2c = x2c + offc
    oy2c = y2c + offc

    area_r = (ox2r - ox1r) * (oy2r - oy1r)
    area_c = (ox2c - ox1c) * (oy2c - oy1c)
    w = jnp.maximum(jnp.minimum(ox2c, ox2r) - jnp.maximum(ox1c, ox1r), 0.0)
    h = jnp.maximum(jnp.minimum(oy2c, oy2r) - jnp.maximum(oy1c, oy1r), 0.0)
    inter = w * h
    union = area_c + area_r - inter
    iou = inter / jnp.maximum(union, 1e-6)

    alive_r = sr > _PRE_T
    alive_c = scl > _PRE_T
    hit = (iou > _NMS_T) & alive_r & alive_c
    prec_cr = (scl > sr) | ((scl == sr) & (ixc < ixr))
    prec_rc = (sr > scl) | ((sr == scl) & (ixr < ixc))
    sup_by_row = hit & prec_cr
    sup_by_col = hit & prec_rc

    def body(carry):
        k_row_f, _ = carry
        k_row = k_row_f > 0.0
        supc = jnp.any(sup_by_col & k_row, axis=1, keepdims=True)
        k_col = alive_c & jnp.logical_not(supc)
        supr = jnp.any(sup_by_row & k_col, axis=0, keepdims=True)
        k_new = alive_r & jnp.logical_not(supr)
        changed = jnp.any(k_new != k_row)
        return (k_new.astype(jnp.float32), changed)

    k_row_f, _ = lax.while_loop(
        lambda c: c[1], body, (alive_r.astype(jnp.float32), jnp.bool_(True)))
    k_row = k_row_f > 0.0
    supc = jnp.any(sup_by_col & k_row, axis=1, keepdims=True)
    k_col = alive_c & jnp.logical_not(supc)

    precf = prec_cr.astype(jnp.float32)
    kcf = k_col.astype(jnp.float32)
    kept_before = jnp.sum(kcf * precf, axis=0, keepdims=True)
    nk_before = jnp.sum((1.0 - kcf) * precf, axis=0, keepdims=True)
    total_kept = jnp.sum(kcf)
    slot = jnp.where(k_row, kept_before, total_kept + nk_before)

    ro = lax.broadcasted_iota(jnp.int32, (_RP, _KP), 0)
    onehot = (ro == slot.astype(jnp.int32)).astype(jnp.float32)
    s_out = jnp.where(k_row, sr, -1.0)

    lane = lax.broadcasted_iota(jnp.int32, (_RP, 128), 1)
    acc = jnp.zeros((_RP, 128), jnp.float32)
    for c, v in enumerate((x1r, y1r, x2r, y2r, s_out)):
        colv = jnp.sum(onehot * v, axis=1, keepdims=True)
        acc = acc + jnp.where(lane == c, colv, 0.0)
    out_ref[...] = acc


def kernel(boxes, scores, labels):
    scores_p = jnp.pad(scores, (0, _NT - _N))
    core = jnp.concatenate(
        [boxes, scores[:, None], labels.astype(jnp.float32)[:, None]], axis=1)
    core_p = jnp.pad(core, ((0, _NT - _N), (0, 0)))
    idx_col = jnp.arange(_NT, dtype=jnp.float32)[:, None]
    table = jnp.pad(jnp.concatenate([core_p, idx_col], axis=1),
                    ((0, 0), (0, 9)))

    g = _sc_select(scores_p, table)
    pc = g
    pr = g[:, :7].T
    bx = boxes.reshape(625, 128)
    out = pl.pallas_call(
        _nms_body,
        out_shape=jax.ShapeDtypeStruct((_RP, 128), jnp.float32),
    )(pr, pc, bx)
    return out[:_OUT, :5]


# speedup vs baseline: 1.0563x; 1.0563x over previous
import functools

import jax
import jax.numpy as jnp
from jax import lax
from jax.experimental import pallas as pl
from jax.experimental.pallas import tpu as pltpu
from jax.experimental.pallas import tpu_sc as plsc

_PRE_T = 0.05
_NMS_T = 0.6
_K = 1000
_KP = 1024
_RP = 104
_OUT = 100

_N = 20000
_NT = 20480
_TILES = 16
_E = _NT // _TILES
_CH = _E // 16
_RUN = _E + 16


def _sc_select_body(scores_hbm, table_hbm, out_hbm,
                    sc_v, key_v, hist_v, allh_v, cnt_v, cnta_v, gt_v, eq_v,
                    sel_v, selg_v, rows_v,
                    sh_hist, sh_cnt, sh_gt, sh_eq, sh_sel, sem):
    sid = lax.axis_index("s")
    base = sid * _E
    lanes = lax.iota(jnp.int32, 16)
    ones_i = jnp.ones((16,), jnp.int32)

    pltpu.sync_copy(scores_hbm.at[pl.ds(base, _E)], sc_v)

    def keys_body(c, _):
        s = sc_v[pl.ds(c * 16, 16)]
        st = jnp.where(s > _PRE_T, s, -1.0)
        k = plsc.bitcast(st, jnp.uint32) ^ jnp.uint32(0x80000000)
        key_v[pl.ds(c * 16, 16)] = k
        return 0

    lax.fori_loop(0, _CH, keys_body, 0)

    rem = jnp.int32(_K)
    prefix = jnp.uint32(0)
    for shift in (24, 16, 8, 0):
        def clr(i, _):
            hist_v[pl.ds(i * 16, 16)] = jnp.zeros((16,), jnp.int32)
            return 0

        lax.fori_loop(0, 16, clr, 0)

        def hb(c, _, shift=shift, prefix=prefix):
            k = key_v[pl.ds(c * 16, 16)]
            digit = ((k >> shift) & jnp.uint32(0xFF)).astype(jnp.int32)
            if shift == 24:
                plsc.addupdate_scatter(hist_v, [digit], ones_i)
            else:
                pm = (k >> (shift + 8)) == (prefix >> (shift + 8))
                plsc.addupdate_scatter(hist_v, [digit], ones_i, mask=pm)
            return 0

        lax.fori_loop(0, _CH, hb, 0)

        pltpu.sync_copy(hist_v, sh_hist.at[sid])
        plsc.subcore_barrier()
        pltpu.sync_copy(sh_hist, allh_v)

        def mg(c, _):
            acc = jnp.zeros((16,), jnp.int32)
            for t in range(_TILES):
                acc = acc + allh_v[t, pl.ds(c * 16, 16)]
            hist_v[pl.ds(c * 16, 16)] = acc
            return 0

        lax.fori_loop(0, 16, mg, 0)
        plsc.subcore_barrier()

        def scan(i, st, rem=rem):
            found, dstar, above = st
            cc = 15 - i
            chunk = hist_v[pl.ds(cc * 16, 16)]
            rev = lax.rev(chunk, (0,))
            cum = jnp.cumsum(rev) + above
            hitv = cum >= rem
            nhit = jnp.sum(hitv.astype(jnp.int32))
            f0 = jnp.min(jnp.where(hitv, lanes, 16))
            excl = cum - rev
            excl_at = jnp.sum(jnp.where(lanes == f0, excl, 0))
            d_new = cc * 16 + 15 - f0
            take = (nhit > 0) & (found == 0)
            found2 = jnp.where(take, 1, found)
            dstar2 = jnp.where(take, d_new, dstar)
            above2 = jnp.where(
                take, excl_at,
                jnp.where(found == 0, above + jnp.sum(chunk), above))
            return (found2, dstar2, above2)

        _, dstar, cnt_gt = lax.fori_loop(
            0, 16, scan, (jnp.int32(0), jnp.int32(0), jnp.int32(0)))
        rem = rem - cnt_gt
        prefix = prefix | (dstar.astype(jnp.uint32) << shift)

    vkey = prefix

    def cp(c, st):
        pg, pe = st
        kk = key_v[pl.ds(c * 16, 16)]
        gidx = base + c * 16 + lanes
        m_gt = kk > vkey
        m_eq = kk == vkey
        plsc.store_compressed(gt_v.at[pl.ds(pg, 16)], gidx, mask=m_gt)
        plsc.store_compressed(eq_v.at[pl.ds(pe, 16)], gidx, mask=m_eq)
        return (pg + jnp.sum(m_gt.astype(jnp.int32)),
                pe + jnp.sum(m_eq.astype(jnp.int32)))

    n_gt, n_eq = lax.fori_loop(0, _CH, cp, (jnp.int32(0), jnp.int32(0)))
    cnt_v[...] = jnp.where(lanes == 0, n_gt, jnp.where(lanes == 1, n_eq, 0))
    pltpu.sync_copy(cnt_v, sh_cnt.at[sid])
    pltpu.sync_copy(gt_v, sh_gt.at[sid])
    pltpu.sync_copy(eq_v, sh_eq.at[sid])
    plsc.subcore_barrier()

    @pl.when(sid == 0)
    def _():
        pltpu.sync_copy(sh_cnt, cnta_v)
        ngt_vec = jnp.zeros((16,), jnp.int32)
        neq_vec = jnp.zeros((16,), jnp.int32)
        for t in range(_TILES):
            row = cnta_v[t, pl.ds(0, 16)]
            ngt_vec = jnp.where(lanes == t, row[0], ngt_vec)
            neq_vec = jnp.where(lanes == t, row[1], neq_vec)
        pos_gt = jnp.cumsum(ngt_vec) - ngt_vec
        total_gt = jnp.sum(ngt_vec)
        excl_eq = jnp.cumsum(neq_vec) - neq_vec
        take_vec = jnp.clip(rem - excl_eq, 0, neq_vec)
        pos_eq = total_gt + jnp.cumsum(take_vec) - take_vec

        def gt_t(t, _):
            n = jnp.sum(jnp.where(lanes == t, ngt_vec, 0))
            pos = jnp.sum(jnp.where(lanes == t, pos_gt, 0))
            pltpu.sync_copy(sh_gt.at[t], gt_v)

            def inner(c, _):
                sel_v[pl.ds(pos + c * 16, 16)] = gt_v[pl.ds(c * 16, 16)]
                return 0

            lax.fori_loop(0, (n + 15) // 16, inner, 0)
            return 0

        lax.fori_loop(0, _TILES, gt_t, 0)

        def eq_t(t, _):
            take = jnp.sum(jnp.where(lanes == t, take_vec, 0))

            @pl.when(take > 0)
            def _():
                pos = jnp.sum(jnp.where(lanes == t, pos_eq, 0))
                pltpu.sync_copy(sh_eq.at[t], eq_v)

                def inner(c, _):
                    sel_v[pl.ds(pos + c * 16, 16)] = eq_v[pl.ds(c * 16, 16)]
                    return 0

                lax.fori_loop(0, (take + 15) // 16, inner, 0)

            return 0

        lax.fori_loop(0, _TILES, eq_t, 0)
        sel_v[pl.ds(_K, 16)] = _N + lanes
        sel_v[pl.ds(_K + 8, 16)] = _N + 8 + lanes
        pltpu.sync_copy(sel_v.at[pl.ds(0, _KP)], sh_sel)

    plsc.subcore_barrier()

    pltpu.sync_copy(sh_sel.at[pl.ds(sid * 64, 64)], selg_v)
    pltpu.async_copy(table_hbm.at[selg_v], rows_v, sem).wait()
    pltpu.sync_copy(rows_v, out_hbm.at[pl.ds(sid * 64, 64)])


_sc_select = functools.partial(
    pl.kernel,
    mesh=plsc.VectorSubcoreMesh(
        core_axis_name="c", subcore_axis_name="s", num_cores=1),
    compiler_params=pltpu.CompilerParams(
        needs_layout_passes=False, use_tc_tiling_on_sc=False),
    out_type=jax.ShapeDtypeStruct((_KP, 16), jnp.float32),
    scratch_types=[
        pltpu.VMEM((_E,), jnp.float32),
        pltpu.VMEM((_E,), jnp.uint32),
        pltpu.VMEM((256,), jnp.int32),
        pltpu.VMEM((_TILES, 256), jnp.int32),
        pltpu.VMEM((16,), jnp.int32),
        pltpu.VMEM((_TILES, 16), jnp.int32),
        pltpu.VMEM((_RUN,), jnp.int32),
        pltpu.VMEM((_RUN,), jnp.int32),
        pltpu.VMEM((_KP + 16,), jnp.int32),
        pltpu.VMEM((64,), jnp.int32),
        pltpu.VMEM((64, 16), jnp.float32),
        pltpu.VMEM_SHARED((_TILES, 256), jnp.int32),
        pltpu.VMEM_SHARED((_TILES, 16), jnp.int32),
        pltpu.VMEM_SHARED((_TILES, _RUN), jnp.int32),
        pltpu.VMEM_SHARED((_TILES, _RUN), jnp.int32),
        pltpu.VMEM_SHARED((_KP,), jnp.int32),
        pltpu.SemaphoreType.DMA,
    ],
)(_sc_select_body)


def _nms_body(pr_ref, pc_ref, bx_ref, out_ref):
    m = jnp.max(bx_ref[...]) + 1.0
    x1r = pr_ref[0:1, :]
    y1r = pr_ref[1:2, :]
    x2r = pr_ref[2:3, :]
    y2r = pr_ref[3:4, :]
    rsr = pr_ref[4:5, :]
    lr = pr_ref[5:6, :]
    ixr = pr_ref[6:7, :]
    x1c = pc_ref[:, 0:1]
    y1c = pc_ref[:, 1:2]
    x2c = pc_ref[:, 2:3]
    y2c = pc_ref[:, 3:4]
    rsc = pc_ref[:, 4:5]
    lc = pc_ref[:, 5:6]
    ixc = pc_ref[:, 6:7]

    sr = jnp.where(rsr > _PRE_T, rsr, -1.0)
    scl = jnp.where(rsc > _PRE_T, rsc, -1.0)

    offr = lr * m
    offc = lc * m
    ox1r = x1r + offr
    oy1r = y1r + offr
    ox2r = x2r + offr
    oy2r = y2r + offr
    ox1c = x1c + offc
    oy1c = y1c + offc
    ox2c = x2c + offc
    oy2c = y2c + offc

    area_r = (ox2r - ox1r) * (oy2r - oy1r)
    area_c = (ox2c - ox1c) * (oy2c - oy1c)
    w = jnp.maximum(jnp.minimum(ox2c, ox2r) - jnp.maximum(ox1c, ox1r), 0.0)
    h = jnp.maximum(jnp.minimum(oy2c, oy2r) - jnp.maximum(oy1c, oy1r), 0.0)
    inter = w * h
    union = area_c + area_r - inter
    iou = inter / jnp.maximum(union, 1e-6)

    alive_r = sr > _PRE_T
    alive_c = scl > _PRE_T
    hit = (iou > _NMS_T) & alive_r & alive_c
    prec_cr = (scl > sr) | ((scl == sr) & (ixc < ixr))
    prec_rc = (sr > scl) | ((sr == scl) & (ixr < ixc))
    sup_by_row = hit & prec_cr
    sup_by_col = hit & prec_rc

    def body(carry):
        k_row_f, _ = carry
        k_row = k_row_f > 0.0
        supc = jnp.any(sup_by_col & k_row, axis=1, keepdims=True)
        k_col = alive_c & jnp.logical_not(supc)
        supr = jnp.any(sup_by_row & k_col, axis=0, keepdims=True)
        k_new = alive_r & jnp.logical_not(supr)
        changed = jnp.any(k_new != k_row)
        return (k_new.astype(jnp.float32), changed)

    k_row_f, _ = lax.while_loop(
        lambda c: c[1], body, (alive_r.astype(jnp.float32), jnp.bool_(True)))
    k_row = k_row_f > 0.0
    supc = jnp.any(sup_by_col & k_row, axis=1, keepdims=True)
    k_col = alive_c & jnp.logical_not(supc)

    precf = prec_cr.astype(jnp.float32)
    kcf = k_col.astype(jnp.float32)
    kept_before = jnp.sum(kcf * precf, axis=0, keepdims=True)
    nk_before = jnp.sum((1.0 - kcf) * precf, axis=0, keepdims=True)
    total_kept = jnp.sum(kcf)
    slot = jnp.where(k_row, kept_before, total_kept + nk_before)

    ro = lax.broadcasted_iota(jnp.int32, (_RP, _KP), 0)
    onehot = (ro == slot.astype(jnp.int32)).astype(jnp.float32)
    s_out = jnp.where(k_row, sr, -1.0)

    lane = lax.broadcasted_iota(jnp.int32, (_RP, 128), 1)
    acc = jnp.zeros((_RP, 128), jnp.float32)
    for c, v in enumerate((x1r, y1r, x2r, y2r, s_out)):
        colv = jnp.sum(onehot * v, axis=1, keepdims=True)
        acc = acc + jnp.where(lane == c, colv, 0.0)
    out_ref[...] = acc


def kernel(boxes, scores, labels):
    scores_p = jnp.pad(scores, (0, _NT - _N))
    core = jnp.concatenate(
        [boxes, scores[:, None], labels.astype(jnp.float32)[:, None]], axis=1)
    core_p = jnp.pad(core, ((0, _NT - _N), (0, 0)))
    idx_col = jnp.arange(_NT, dtype=jnp.float32)[:, None]
    table = jnp.pad(jnp.concatenate([core_p, idx_col], axis=1),
                    ((0, 0), (0, 9)))

    g = _sc_select(scores_p, table)
    pc = g
    pr = g[:, :7].T
    bx = boxes.reshape(625, 128)
    out = pl.pallas_call(
        _nms_body,
        out_shape=jax.ShapeDtypeStruct((_RP, 128), jnp.float32),
    )(pr, pc, bx)
    return out[:_OUT, :5]
